# baseline (speedup 1.0000x reference)
import functools

import jax
import jax.numpy as jnp
from jax import lax
from jax.experimental import pallas as pl
from jax.experimental.pallas import tpu as pltpu
from jax.experimental.pallas import tpu_sc as plsc

BATCH = 4096
ENT = 100
DIM = 64
NC, NS = 2, 16
NW = NC * NS
NVW = 21
NMW = NW - NVW
CB = 128
NCK = BATCH // CB
DH, DL, BH, BL = DIM // 8, 8, BATCH // 128, 128
PPITCH = 129


def _sc_body(xt_hbm, tab_hbm, out_hbm, xrow, cidx, gb0, gb1, pb0, pb1, slab,
             gs0, gs1, os0, os1, msem, msem2):
    wid = lax.axis_index("s") * NC + lax.axis_index("c")

    vk = jnp.where(wid < NVW, wid, wid - NVW)
    cls_w = jnp.where(wid < NVW, NVW, NMW)
    nplanes = jnp.where(vk < ENT % cls_w, ENT // cls_w + 1, ENT // cls_w)

    gbufs = ((gb0, gs0, os0), (gb1, gs1, os1))

    def fire_gathers(k, buf, gsem):
        return pltpu.async_copy(
            tab_hbm.at[cidx.at[pl.ds(k * CB, CB)]], buf, gsem)

    @pl.when(wid < NVW)
    def _vec_planes():
        def plane(i, carry):
            e = vk + NVW * i
            pltpu.sync_copy(xt_hbm.at[e], xrow)

            @plsc.parallel_loop(0, BATCH // 16, unroll=4)
            def _clean(j):
                v = xrow[pl.ds(j * 16, 16)]
                cidx[pl.ds(j * 16, 16)] = jnp.where(v < 0, 0, v)

            fire_gathers(0, gb0, gs0)

            def chunk2(k2, c2):
                for bi in range(2):
                    k = k2 * 2 + bi
                    buf, gsem, osem = gbufs[bi]
                    pbuf = pb0 if bi == 0 else pb1

                    @pl.when(k2 * 2 + bi < NCK - 1)
                    def _next():
                        nbuf, ngsem, _ = gbufs[1 - bi]
                        fire_gathers(k + 1, nbuf, ngsem)

                    pltpu.make_async_copy(
                        tab_hbm.at[cidx.at[pl.ds(0, CB)]], buf, gsem).wait()

                    @pl.when(c2 + bi >= 2)
                    def _drain():
                        pltpu.make_async_copy(
                            pbuf.at[:, :, :, pl.ds(0, BL)],
                            out_hbm.at[0, 0, :, pl.ds(0, 1)],
                            osem).wait()

                    lane = lax.iota(jnp.int32, 16)
                    dlv = lane % 8
                    zv = jnp.zeros((16,), jnp.int32)

                    @plsc.parallel_loop(0, CB, unroll=2)
                    def _transpose(b):
                        blv = jnp.full((16,), b, jnp.int32)
                        for kd in range(DIM // 16):
                            v = buf[b, pl.ds(kd * 16, 16)]
                            dhv = 2 * kd + lane // 8
                            plsc.store_scatter(pbuf, [dhv, zv, dlv, blv], v)

                    pltpu.async_copy(
                        pbuf.at[:, :, :, pl.ds(0, BL)],
                        out_hbm.at[0, e, :, pl.ds(k, 1)],
                        osem)
                return c2 + 2

            return lax.fori_loop(0, NCK // 2, chunk2, carry)

        total = lax.fori_loop(0, nplanes, plane, 0)

        @pl.when(total > 0)
        def _final_drain():
            for pbuf, osem in ((pb0, os0), (pb1, os1)):
                pltpu.make_async_copy(
                    pbuf.at[:, :, :, pl.ds(0, BL)],
                    out_hbm.at[0, 0, :, pl.ds(0, 1)], osem).wait()

    @pl.when(wid >= NVW)
    def _mask_planes():
        msems = (msem, msem2)

        def _dr(sem):
            for dh in range(DH):
                pltpu.make_async_copy(slab.at[0], out_hbm.at[1, 0, 0],
                                      sem).wait()

        def plane_pair(i2, carry):
            for par in range(2):
                i = i2 * 2 + par
                sem = msems[par]

                @pl.when(i < nplanes)
                def _do():
                    e = vk + NMW * i

                    @pl.when(i >= 2)
                    def _drain_prev():
                        _dr(sem)

                    pltpu.sync_copy(xt_hbm.at[e], xrow)

                    @plsc.parallel_loop(0, BH, unroll=2)
                    def _bh_body(bh):
                        for blg in range(8):
                            v = xrow[pl.ds(bh * 128 + blg * 16, 16)]
                            m = jnp.where(v < 0, 0.0,
                                          1.0).astype(jnp.float32)
                            for dl in range(DL):
                                slab[par, bh, dl, pl.ds(blg * 16, 16)] = m

                    for dh in range(DH):
                        pltpu.async_copy(slab.at[par], out_hbm.at[1, e, dh],
                                         sem)
            return carry + 2

        lax.fori_loop(0, (ENT // NMW + 2) // 2, plane_pair, 0)

        @pl.when(nplanes > 0)
        def _final_drain1():
            _dr(msem)

        @pl.when(nplanes > 1)
        def _final_drain2():
            _dr(msem2)


_sc_call = functools.partial(
    pl.kernel,
    out_type=jax.ShapeDtypeStruct((2, ENT, DH, BH, DL, BL), jnp.float32),
    mesh=plsc.VectorSubcoreMesh(core_axis_name="c", subcore_axis_name="s",
                                num_cores=NC, num_subcores=NS),
    scratch_types=[
        pltpu.VMEM((BATCH,), jnp.int32),
        pltpu.VMEM((BATCH,), jnp.int32),
        pltpu.VMEM((CB, DIM), jnp.float32),
        pltpu.VMEM((CB, DIM), jnp.float32),
        pltpu.VMEM((DH, 1, DL, PPITCH), jnp.float32),
        pltpu.VMEM((DH, 1, DL, PPITCH), jnp.float32),
        pltpu.VMEM((2, BH, DL, BL), jnp.float32),
        pltpu.SemaphoreType.DMA,
        pltpu.SemaphoreType.DMA,
        pltpu.SemaphoreType.DMA,
        pltpu.SemaphoreType.DMA,
        pltpu.SemaphoreType.DMA,
        pltpu.SemaphoreType.DMA,
    ],
    compiler_params=pltpu.CompilerParams(use_tc_tiling_on_sc=False,
                                         needs_layout_passes=False),
)(_sc_body)


def kernel(x, entity_vectors):
    out6 = _sc_call(x.T, entity_vectors)
    return jnp.transpose(out6, (3, 5, 0, 1, 2, 4)).reshape(BATCH, 2, ENT, DIM)
